# baseline (speedup 1.0000x reference)
import jax
import jax.numpy as jnp
import numpy as np
from jax import lax
from jax.experimental import pallas as pl
from jax.experimental.pallas import tpu as pltpu
from jax.experimental.pallas import tpu_sc as plsc

V = 100000
D = 128
C = 4
CP = 16
B = 4096
S = 200
H0 = 104
H1 = S - H0

NC = 2
NS = 16
NW = NC * NS
BW = B // NW
NBUF = 4

FOLD_R = 20000


def _fold_body(tbl_ref, w1_ref, w2_ref, psel_ref, qsel_ref, msk_ref, out_ref):
    w2p = jnp.concatenate(
        [w2_ref[...], jnp.zeros((CP - C, D), jnp.float32)], axis=0)
    wct = lax.dot_general(w1_ref[...], w2p, (((0,), (1,)), ((), ())),
                          preferred_element_type=jnp.float32)
    a = lax.dot_general(psel_ref[...], wct, (((1,), (0,)), ((), ())),
                        preferred_element_type=jnp.float32)
    a = lax.dot_general(a, qsel_ref[...], (((1,), (0,)), ((), ())),
                        preferred_element_type=jnp.float32)
    bd = a * msk_ref[...]
    tbl8 = jnp.reshape(tbl_ref[...], (FOLD_R // 8, 8 * D))
    out_ref[...] = jnp.reshape(
        lax.dot_general(tbl8, bd, (((1,), (0,)), ((), ())),
                        preferred_element_type=jnp.float32),
        (1, FOLD_R // 8, D))


def _fold(table, w1, w2):
    grid = V // FOLD_R
    m8 = FOLD_R // 8
    q_i = np.arange(8 * D, dtype=np.int64)
    l_i = np.arange(D, dtype=np.int64)
    c_i = np.arange(CP, dtype=np.int64)
    psel = jnp.asarray((q_i[:, None] % D == l_i[None, :]), jnp.float32)
    qsel = jnp.asarray((l_i[None, :] % CP == c_i[:, None]), jnp.float32)
    msk = jnp.asarray((q_i[:, None] // D == l_i[None, :] // CP), jnp.float32)
    return pl.pallas_call(
        _fold_body,
        grid=(grid,),
        in_specs=[
            pl.BlockSpec((FOLD_R, D), lambda i: (i, 0)),
            pl.BlockSpec((D, D), lambda i: (0, 0)),
            pl.BlockSpec((C, D), lambda i: (0, 0)),
            pl.BlockSpec((8 * D, D), lambda i: (0, 0)),
            pl.BlockSpec((CP, D), lambda i: (0, 0)),
            pl.BlockSpec((8 * D, D), lambda i: (0, 0)),
        ],
        out_specs=pl.BlockSpec((1, m8, D), lambda i: (i, 0, 0)),
        out_shape=jax.ShapeDtypeStruct((grid, m8, D), jnp.float32),
    )(table, w1, w2, psel, qsel, msk)


def _sc_body(x_hbm, t2_hbm, zsum_hbm, idx_v, rows_v, out_v, *sems):
    wid = lax.axis_index("s") * NC + lax.axis_index("c")
    base = wid * BW

    pltpu.sync_copy(x_hbm.at[pl.ds(base, BW)], idx_v)

    def issue(b, buf):
        pltpu.async_copy(t2_hbm.at[idx_v.at[b]], rows_v.at[buf], sems[buf])

    def issue_if(b, buf):
        @pl.when(b < BW)
        def _():
            issue(b, buf)

    def drain(b, buf):
        pltpu.make_async_copy(
            t2_hbm.at[idx_v.at[b]], rows_v.at[buf], sems[buf]).wait()

    def accum(b, buf):
        zero = jnp.zeros((CP,), jnp.float32)
        a = [zero] * 16
        for r in range(S):
            a[r % 16] = a[r % 16] + rows_v[buf, r, :]
        for step in (8, 4, 2, 1):
            for k in range(step):
                a[k] = a[k] + a[k + step]
        out_v[b, :] = a[0]

    for p in range(NBUF - 1):
        issue(p, p)

    def outer(i, carry):
        b0 = NBUF * i
        for k in range(NBUF):
            issue_if(b0 + k + NBUF - 1, (k + NBUF - 1) % NBUF)
            drain(b0 + k, k)
            accum(b0 + k, k)
        return carry

    lax.fori_loop(0, BW // NBUF, outer, 0)
    pltpu.sync_copy(out_v, zsum_hbm.at[pl.ds(base, BW)])


def _sc_gather_sum(x3, t2):
    mesh = plsc.VectorSubcoreMesh(core_axis_name="c", subcore_axis_name="s")
    return pl.kernel(
        _sc_body,
        mesh=mesh,
        compiler_params=pltpu.CompilerParams(use_tc_tiling_on_sc=False),
        out_type=jax.ShapeDtypeStruct((B, CP), jnp.float32),
        scratch_types=[
            pltpu.VMEM((BW, S), jnp.int32),
            pltpu.VMEM((NBUF, S, CP), jnp.float32),
            pltpu.VMEM((BW, CP), jnp.float32),
        ] + [pltpu.SemaphoreType.DMA] * NBUF,
    )(x3, t2)


def _head_body(zs_ref, w2_ref, b1_ref, b2_ref, out_ref):
    bc = lax.dot_general(b1_ref[...], w2_ref[...], (((1,), (1,)), ((), ())),
                         preferred_element_type=jnp.float32)
    bc = bc + b2_ref[...]
    z = zs_ref[:, 0:C] * jnp.float32(1.0 / S) + bc
    mx = jnp.max(z, axis=1, keepdims=True)
    lse = jnp.log(jnp.sum(jnp.exp(z - mx), axis=1, keepdims=True)) + mx
    out_ref[...] = z - lse


def _head(zsum, w2, b1r, b2r):
    return pl.pallas_call(
        _head_body,
        out_shape=jax.ShapeDtypeStruct((B, C), jnp.float32),
    )(zsum, w2, b1r, b2r)


def kernel(x, table, W1, b1, W2, b2):
    x3 = x.astype(jnp.int32)
    b1r = b1.reshape(1, D)
    b2r = b2.reshape(1, C)
    t2 = _fold(table, W1, W2).reshape(V, CP)
    zsum = _sc_gather_sum(x3, t2)
    return _head(zsum, W2, b1r, b2r)


# speedup vs baseline: 1.0278x; 1.0278x over previous
import jax
import jax.numpy as jnp
import numpy as np
from jax import lax
from jax.experimental import pallas as pl
from jax.experimental.pallas import tpu as pltpu
from jax.experimental.pallas import tpu_sc as plsc

V = 100000
D = 128
C = 4
CP = 16
B = 4096
S = 200
H0 = 104
H1 = S - H0

NC = 2
NS = 16
NW = NC * NS
BW = B // NW
NBUF = 4

FOLD_R = 20000


def _fold_body(tbl_ref, w1_ref, w2_ref, psel_ref, qsel_ref, msk_ref, out_ref):
    w2p = jnp.concatenate(
        [w2_ref[...], jnp.zeros((CP - C, D), jnp.float32)], axis=0)
    wct = lax.dot_general(w1_ref[...], w2p, (((0,), (1,)), ((), ())),
                          preferred_element_type=jnp.float32)
    a = lax.dot_general(psel_ref[...], wct, (((1,), (0,)), ((), ())),
                        preferred_element_type=jnp.float32)
    a = lax.dot_general(a, qsel_ref[...], (((1,), (0,)), ((), ())),
                        preferred_element_type=jnp.float32)
    bd = a * msk_ref[...]
    tbl8 = jnp.reshape(tbl_ref[...], (FOLD_R // 8, 8 * D))
    out_ref[...] = jnp.reshape(
        lax.dot_general(tbl8, bd, (((1,), (0,)), ((), ())),
                        preferred_element_type=jnp.float32),
        (1, FOLD_R // 8, D))


def _fold(table, w1, w2):
    grid = V // FOLD_R
    m8 = FOLD_R // 8
    q_i = np.arange(8 * D, dtype=np.int64)
    l_i = np.arange(D, dtype=np.int64)
    c_i = np.arange(CP, dtype=np.int64)
    psel = jnp.asarray((q_i[:, None] % D == l_i[None, :]), jnp.float32)
    qsel = jnp.asarray((l_i[None, :] % CP == c_i[:, None]), jnp.float32)
    msk = jnp.asarray((q_i[:, None] // D == l_i[None, :] // CP), jnp.float32)
    return pl.pallas_call(
        _fold_body,
        grid=(grid,),
        in_specs=[
            pl.BlockSpec((FOLD_R, D), lambda i: (i, 0)),
            pl.BlockSpec((D, D), lambda i: (0, 0)),
            pl.BlockSpec((C, D), lambda i: (0, 0)),
            pl.BlockSpec((8 * D, D), lambda i: (0, 0)),
            pl.BlockSpec((CP, D), lambda i: (0, 0)),
            pl.BlockSpec((8 * D, D), lambda i: (0, 0)),
        ],
        out_specs=pl.BlockSpec((1, m8, D), lambda i: (i, 0, 0)),
        out_shape=jax.ShapeDtypeStruct((grid, m8, D), jnp.float32),
    )(table, w1, w2, psel, qsel, msk)


def _sc_body(x_hbm, t2_hbm, zsum_hbm, idx_v, rows_v, out_v, *sems):
    wid = lax.axis_index("s") * NC + lax.axis_index("c")
    base = wid * BW

    pltpu.sync_copy(x_hbm.at[pl.ds(base, BW)], idx_v)

    def issue(b, buf):
        pltpu.async_copy(t2_hbm.at[idx_v.at[b]], rows_v.at[buf], sems[buf])

    def issue_if(b, buf):
        @pl.when(b < BW)
        def _():
            issue(b, buf)

    def drain(b, buf):
        pltpu.make_async_copy(
            t2_hbm.at[idx_v.at[b]], rows_v.at[buf], sems[buf]).wait()

    def accum(b, buf):
        zero = jnp.zeros((CP,), jnp.float32)

        def body(q, a):
            r0 = 8 * q
            return tuple(a[k] + rows_v[buf, r0 + k, :] for k in range(8))

        a = lax.fori_loop(0, S // 8, body, (zero,) * 8)
        t0 = (a[0] + a[1]) + (a[2] + a[3])
        t1 = (a[4] + a[5]) + (a[6] + a[7])
        out_v[b, :] = t0 + t1

    for p in range(NBUF - 1):
        issue(p, p)

    def outer(i, carry):
        b0 = NBUF * i
        for k in range(NBUF):
            issue_if(b0 + k + NBUF - 1, (k + NBUF - 1) % NBUF)
            drain(b0 + k, k)
            accum(b0 + k, k)
        return carry

    lax.fori_loop(0, BW // NBUF, outer, 0)
    pltpu.sync_copy(out_v, zsum_hbm.at[pl.ds(base, BW)])


def _sc_gather_sum(x3, t2):
    mesh = plsc.VectorSubcoreMesh(core_axis_name="c", subcore_axis_name="s")
    return pl.kernel(
        _sc_body,
        mesh=mesh,
        compiler_params=pltpu.CompilerParams(use_tc_tiling_on_sc=False),
        out_type=jax.ShapeDtypeStruct((B, CP), jnp.float32),
        scratch_types=[
            pltpu.VMEM((BW, S), jnp.int32),
            pltpu.VMEM((NBUF, S, CP), jnp.float32),
            pltpu.VMEM((BW, CP), jnp.float32),
        ] + [pltpu.SemaphoreType.DMA] * NBUF,
    )(x3, t2)


def _head_body(zs_ref, w2_ref, b1_ref, b2_ref, out_ref):
    bc = lax.dot_general(b1_ref[...], w2_ref[...], (((1,), (1,)), ((), ())),
                         preferred_element_type=jnp.float32)
    bc = bc + b2_ref[...]
    z = zs_ref[:, 0:C] * jnp.float32(1.0 / S) + bc
    mx = jnp.max(z, axis=1, keepdims=True)
    lse = jnp.log(jnp.sum(jnp.exp(z - mx), axis=1, keepdims=True)) + mx
    out_ref[...] = z - lse


def _head(zsum, w2, b1r, b2r):
    return pl.pallas_call(
        _head_body,
        out_shape=jax.ShapeDtypeStruct((B, C), jnp.float32),
    )(zsum, w2, b1r, b2r)


def kernel(x, table, W1, b1, W2, b2):
    x3 = x.astype(jnp.int32)
    b1r = b1.reshape(1, D)
    b2r = b2.reshape(1, C)
    t2 = _fold(table, W1, W2).reshape(V, CP)
    zsum = _sc_gather_sum(x3, t2)
    return _head(zsum, W2, b1r, b2r)
